# baseline (speedup 1.0000x reference)
import jax
import jax.numpy as jnp
from jax import lax
from jax.experimental import pallas as pl
from jax.experimental.pallas import tpu as pltpu
from jax.experimental.pallas import tpu_sc as plsc

_B = 16384
_D = 32
_NEG = 5
_NC = 2
_NS = 16
_NW = _NC * _NS
_L = 16
_BPW = _B // _NW
_CHUNK = 128
_UC = _BPW // _CHUNK
_JC = _BPW * _NEG // _CHUNK
_NGROUP = _BPW // _L
_DW = _D // 2


def _log_sigmoid(x):
    e = jnp.exp(-jnp.abs(x))
    z = e / (e + 2.0)
    z2 = z * z
    p = 1.0 + z2 * (1.0 / 3.0 + z2 * (1.0 / 5.0 + z2 * (
        1.0 / 7.0 + z2 * (1.0 / 9.0 + z2 * (1.0 / 11.0 + z2 * (1.0 / 13.0))))))
    return jnp.minimum(x, 0.0) - 2.0 * z * p


def _sc_body(vertex_hbm, context_hbm, u_hbm, i_hbm, js_hbm, out_hbm,
             u_idx, i_idx, js_idx, v1, v2, ng, acc_buf, sem):
    c = lax.axis_index("c")
    s = lax.axis_index("s")
    wid = s * _NC + c

    pltpu.sync_copy(u_hbm.at[wid], u_idx)
    pltpu.sync_copy(i_hbm.at[wid], i_idx)
    pltpu.sync_copy(js_hbm.at[wid], js_idx)

    copies = []
    for k in range(_UC):
        copies.append(pltpu.async_copy(
            vertex_hbm.at[u_idx.at[k]], v1.at[pl.ds(k * _CHUNK, _CHUNK)], sem))
        copies.append(pltpu.async_copy(
            context_hbm.at[i_idx.at[k]],
            v2.at[pl.ds(k * _CHUNK, _CHUNK)], sem))
    for k in range(_JC):
        copies.append(pltpu.async_copy(
            context_hbm.at[js_idx.at[k]],
            ng.at[pl.ds(k * _CHUNK, _CHUNK)], sem))
    for cp in copies:
        cp.wait()

    lanes = lax.iota(jnp.int32, _L)

    def _unpack(w):
        lo = plsc.bitcast(jnp.left_shift(w, 16), jnp.float32)
        hi = plsc.bitcast(jnp.left_shift(jnp.right_shift(w, 16), 16),
                          jnp.float32)
        return lo, hi

    def group_body(g, acc):
        rowb = g * _L + lanes
        rown = rowb * _NEG
        pos = jnp.zeros((_L,), jnp.float32)
        negs = [jnp.zeros((_L,), jnp.float32) for _ in range(_NEG)]
        for d in range(_DW):
            dv = (lanes + d) & (_DW - 1)
            alo, ahi = _unpack(plsc.load_gather(v1, [rowb, dv]))
            blo, bhi = _unpack(plsc.load_gather(v2, [rowb, dv]))
            pos = pos + alo * blo + ahi * bhi
            for n in range(_NEG):
                clo, chi = _unpack(plsc.load_gather(ng, [rown + n, dv]))
                negs[n] = negs[n] + alo * clo + ahi * chi
        tot = _log_sigmoid(pos)
        for n in range(_NEG):
            tot = tot + _log_sigmoid(-negs[n])
        return acc + tot

    acc = lax.fori_loop(0, _NGROUP, group_body, jnp.zeros((_L,), jnp.float32))
    acc_buf[...] = acc
    pltpu.sync_copy(acc_buf, out_hbm.at[pl.ds(wid * _L, _L)])


_sc_gather_loss = pl.kernel(
    _sc_body,
    mesh=plsc.VectorSubcoreMesh(core_axis_name="c", subcore_axis_name="s"),
    out_type=jax.ShapeDtypeStruct((_NW * _L,), jnp.float32),
    compiler_params=pltpu.CompilerParams(
        needs_layout_passes=False,
        use_tc_tiling_on_sc=False,
    ),
    scratch_types=[
        pltpu.VMEM((_UC, _CHUNK), jnp.int32),
        pltpu.VMEM((_UC, _CHUNK), jnp.int32),
        pltpu.VMEM((_JC, _CHUNK), jnp.int32),
        pltpu.VMEM((_BPW, _DW), jnp.int32),
        pltpu.VMEM((_BPW, _DW), jnp.int32),
        pltpu.VMEM((_BPW * _NEG, _DW), jnp.int32),
        pltpu.VMEM((_L,), jnp.float32),
        pltpu.SemaphoreType.DMA,
    ],
)


def _finish_body(p_ref, o_ref):
    o_ref[...] = jnp.broadcast_to(-jnp.sum(p_ref[...]) * (1.0 / _B), (1, 1))


_finish = pl.pallas_call(
    _finish_body,
    out_shape=jax.ShapeDtypeStruct((1, 1), jnp.float32),
)


def _pack_table(t):
    return jax.lax.bitcast_convert_type(
        t.astype(jnp.bfloat16).reshape(t.shape[0], _DW, 2), jnp.int32)


def kernel(u, i, js, vertex_emb, context_emb):
    u2 = u.astype(jnp.int32).reshape(_NW, _UC, _CHUNK)
    i2 = i.astype(jnp.int32).reshape(_NW, _UC, _CHUNK)
    js2 = js.astype(jnp.int32).reshape(_NW, _JC, _CHUNK)
    partials = _sc_gather_loss(_pack_table(vertex_emb),
                               _pack_table(context_emb), u2, i2, js2)
    return _finish(partials.reshape(1, _NW * _L))[0, 0]


# speedup vs baseline: 2.1850x; 2.1850x over previous
import jax
import jax.numpy as jnp
from jax import lax
from jax.experimental import pallas as pl
from jax.experimental.pallas import tpu as pltpu
from jax.experimental.pallas import tpu_sc as plsc

_B = 16384
_D = 32
_NEG = 5
_NC = 2
_NS = 16
_NW = _NC * _NS
_L = 16
_BPW = _B // _NW
_CHUNK = 128
_UC = _BPW // _CHUNK
_JC = _BPW * _NEG // _CHUNK
_NGROUP = _BPW // _L


def _log_sigmoid(x):
    e = jnp.exp(-jnp.abs(x))
    z = e / (e + 2.0)
    z2 = z * z
    p = 1.0 + z2 * (1.0 / 3.0 + z2 * (1.0 / 5.0 + z2 * (
        1.0 / 7.0 + z2 * (1.0 / 9.0 + z2 * (1.0 / 11.0 + z2 * (1.0 / 13.0))))))
    return jnp.minimum(x, 0.0) - 2.0 * z * p


def _sc_body(vertex_hbm, context_hbm, u_hbm, i_hbm, js_hbm, out_hbm,
             u_idx, i_idx, js_idx, v1, v2, ng, acc_buf, sem):
    c = lax.axis_index("c")
    s = lax.axis_index("s")
    wid = s * _NC + c

    pltpu.sync_copy(u_hbm.at[wid], u_idx)
    pltpu.sync_copy(i_hbm.at[wid], i_idx)
    pltpu.sync_copy(js_hbm.at[wid], js_idx)

    copies = []
    for k in range(_UC):
        copies.append(pltpu.async_copy(
            vertex_hbm.at[u_idx.at[k]], v1.at[pl.ds(k * _CHUNK, _CHUNK)], sem))
        copies.append(pltpu.async_copy(
            context_hbm.at[i_idx.at[k]],
            v2.at[pl.ds(k * _CHUNK, _CHUNK)], sem))
    for k in range(_JC):
        copies.append(pltpu.async_copy(
            context_hbm.at[js_idx.at[k]],
            ng.at[pl.ds(k * _CHUNK, _CHUNK)], sem))
    for cp in copies:
        cp.wait()

    lanes = lax.iota(jnp.int32, _L)

    def group_body(g, acc):
        rowb = g * _L + lanes
        rown = rowb * _NEG
        pos = jnp.zeros((_L,), jnp.float32)
        negs = [jnp.zeros((_L,), jnp.float32) for _ in range(_NEG)]
        for d in range(_D):
            dv = (lanes + d) & (_D - 1)
            a = plsc.load_gather(v1, [rowb, dv])
            b = plsc.load_gather(v2, [rowb, dv])
            pos = pos + a * b
            for n in range(_NEG):
                cn = plsc.load_gather(ng, [rown + n, dv])
                negs[n] = negs[n] + a * cn
        tot = _log_sigmoid(pos)
        for n in range(_NEG):
            tot = tot + _log_sigmoid(-negs[n])
        return acc + tot

    acc = lax.fori_loop(0, _NGROUP, group_body, jnp.zeros((_L,), jnp.float32))
    acc_buf[...] = acc
    pltpu.sync_copy(acc_buf, out_hbm.at[pl.ds(wid * _L, _L)])


_sc_gather_loss = pl.kernel(
    _sc_body,
    mesh=plsc.VectorSubcoreMesh(core_axis_name="c", subcore_axis_name="s"),
    out_type=jax.ShapeDtypeStruct((_NW * _L,), jnp.float32),
    compiler_params=pltpu.CompilerParams(
        needs_layout_passes=False,
        use_tc_tiling_on_sc=False,
    ),
    scratch_types=[
        pltpu.VMEM((_UC, _CHUNK), jnp.int32),
        pltpu.VMEM((_UC, _CHUNK), jnp.int32),
        pltpu.VMEM((_JC, _CHUNK), jnp.int32),
        pltpu.VMEM((_BPW, _D), jnp.float32),
        pltpu.VMEM((_BPW, _D), jnp.float32),
        pltpu.VMEM((_BPW * _NEG, _D), jnp.float32),
        pltpu.VMEM((_L,), jnp.float32),
        pltpu.SemaphoreType.DMA,
    ],
)


def _finish_body(p_ref, o_ref):
    o_ref[...] = jnp.broadcast_to(-jnp.sum(p_ref[...]) * (1.0 / _B), (1, 1))


_finish = pl.pallas_call(
    _finish_body,
    out_shape=jax.ShapeDtypeStruct((1, 1), jnp.float32),
)


def kernel(u, i, js, vertex_emb, context_emb):
    u2 = u.astype(jnp.int32).reshape(_NW, _UC, _CHUNK)
    i2 = i.astype(jnp.int32).reshape(_NW, _UC, _CHUNK)
    js2 = js.astype(jnp.int32).reshape(_NW, _JC, _CHUNK)
    partials = _sc_gather_loss(vertex_emb, context_emb, u2, i2, js2)
    return _finish(partials.reshape(1, _NW * _L))[0, 0]
